# baseline (speedup 1.0000x reference)
import numpy as np
import jax
import jax.numpy as jnp
from jax.experimental import pallas as pl
from jax.experimental.pallas import tpu as pltpu

_BIN_COUNT = 10
_EDGES = tuple(float(np.arctanh((k - 0.5) / _BIN_COUNT))
               for k in range(_BIN_COUNT + 1))

_VMEM_LIMIT = 48 << 20


def _one_batch(w, k, tri, band, xf):
    c, hw = xf.shape
    n = hw

    mean = jnp.sum(xf, axis=1, keepdims=True) * (1.0 / float(n))
    d = xf - mean
    var = jnp.sum(d * d, axis=1, keepdims=True) * (1.0 / float(max(n - 1, 1)))

    s = [jnp.sum((xf >= t).astype(jnp.float32), axis=1, keepdims=True)
         for t in _EDGES]
    total = s[0]
    c_logc = jnp.zeros_like(total)
    for i in range(_BIN_COUNT + 1):
        ck = (s[i] - s[i + 1]) if i < _BIN_COUNT else s[i]
        c_logc = c_logc + ck * jnp.log(jnp.where(ck > 0, ck, 1.0))
    ent = jnp.log(total) - c_logc / total
    score = ent + 2.0 / (var + 1e-7)

    ones_col = jnp.ones((c, 1), jnp.float32)
    dn_1_1 = (((1,), (1,)), ((), ()))
    srow_b = jax.lax.dot_general(ones_col, score, dn_1_1,
                                 precision=jax.lax.Precision.HIGHEST)
    scol_b = jax.lax.dot_general(score, ones_col, dn_1_1,
                                 precision=jax.lax.Precision.HIGHEST)
    beats = ((srow_b > scol_b).astype(jnp.float32)
             + (srow_b == scol_b).astype(jnp.float32) * tri)
    rank = jnp.dot(beats, ones_col,
                   preferred_element_type=jnp.float32)
    m = (rank < float(k)).astype(jnp.float32)

    ns = jnp.dot(xf.astype(jnp.bfloat16), band,
                 preferred_element_type=jnp.float32)

    return xf + m * (8.0 * xf - ns)


def _fused_kernel(w, k, nb, tri_ref, band_ref, x_ref, o_ref):
    tri = tri_ref[...]
    band = band_ref[...]
    for ib in range(nb):
        xf = x_ref[ib]
        o_ref[ib] = _one_batch(w, k, tri, band, xf).astype(o_ref.dtype)


def kernel(x, rate=0.1):
    b, c, h, w = x.shape
    hw = h * w
    k = int(round(rate * c))
    if k <= 0:
        return x
    x2 = x.reshape(b, c, hw)
    nb = 4 if b % 4 == 0 else 1
    tri = jnp.asarray(np.tri(c, c, -1, dtype=np.float32))
    bh = (np.abs(np.arange(h)[:, None] - np.arange(h)[None, :]) <= 1)
    bw = (np.abs(np.arange(w)[:, None] - np.arange(w)[None, :]) <= 1)
    band = jnp.asarray(np.kron(bh, bw).astype(np.float32),
                       dtype=jnp.bfloat16)
    out2 = pl.pallas_call(
        lambda tri_ref, band_ref, x_ref, o_ref: _fused_kernel(
            w, k, nb, tri_ref, band_ref, x_ref, o_ref),
        out_shape=jax.ShapeDtypeStruct((b, c, hw), x.dtype),
        grid=(b // nb,),
        in_specs=[pl.BlockSpec((c, c), lambda i: (0, 0)),
                  pl.BlockSpec((hw, hw), lambda i: (0, 0)),
                  pl.BlockSpec((nb, c, hw), lambda i: (i, 0, 0))],
        out_specs=pl.BlockSpec((nb, c, hw), lambda i: (i, 0, 0)),
        compiler_params=pltpu.CompilerParams(
            dimension_semantics=("arbitrary",),
            vmem_limit_bytes=_VMEM_LIMIT),
    )(tri, band, x2)
    return out2.reshape(b, c, h, w)


# speedup vs baseline: 1.0643x; 1.0643x over previous
import numpy as np
import jax
import jax.numpy as jnp
from jax.experimental import pallas as pl
from jax.experimental.pallas import tpu as pltpu

_BIN_COUNT = 10
_EDGES = tuple(float(np.arctanh((k - 0.5) / _BIN_COUNT))
               for k in range(_BIN_COUNT + 1))

_VMEM_LIMIT = 48 << 20


def _fused_kernel(w, k, nb, c, tri_ref, band_ref, x_ref, o_ref):
    hw = x_ref.shape[2]
    n = hw
    xf = x_ref[...].reshape(nb * c, hw)

    ch = 128
    chunk_scores = []
    for r0 in range(0, nb * c, ch):
        xc = xf[r0:r0 + ch]
        mean = jnp.sum(xc, axis=1, keepdims=True) * (1.0 / float(n))
        d = xc - mean
        var = jnp.sum(d * d, axis=1, keepdims=True) * (1.0 / float(max(n - 1, 1)))
        s = [jnp.sum((xc >= t).astype(jnp.float32), axis=1, keepdims=True)
             for t in _EDGES]
        scat = jnp.concatenate(s + [jnp.zeros_like(s[0])], axis=1)
        ck = scat[:, :_BIN_COUNT + 1] - scat[:, 1:]
        c_logc = jnp.sum(ck * jnp.log(jnp.where(ck > 0, ck, 1.0)),
                         axis=1, keepdims=True)
        total = s[0]
        ent = jnp.log(total) - c_logc / total
        chunk_scores.append(ent + 2.0 / (var + 1e-7))
    score_all = jnp.concatenate(chunk_scores, axis=0)

    ns = jnp.dot(xf.astype(jnp.bfloat16), band_ref[...],
                 preferred_element_type=jnp.float32)

    tri = tri_ref[...]
    ones_col = jnp.ones((c, 1), jnp.float32)
    dn_1_1 = (((1,), (1,)), ((), ()))
    masks = []
    for ib in range(nb):
        score = score_all[ib * c:(ib + 1) * c]
        srow_b = jax.lax.dot_general(ones_col, score, dn_1_1,
                                     precision=jax.lax.Precision.HIGHEST)
        scol_b = jax.lax.dot_general(score, ones_col, dn_1_1,
                                     precision=jax.lax.Precision.HIGHEST)
        beats = ((srow_b > scol_b).astype(jnp.float32)
                 + (srow_b == scol_b).astype(jnp.float32) * tri)
        rank = jnp.dot(beats, ones_col,
                       preferred_element_type=jnp.float32)
        masks.append((rank < float(k)).astype(jnp.float32))
    m = jnp.concatenate(masks, axis=0)

    out = xf + m * (8.0 * xf - ns)
    o_ref[...] = out.reshape(nb, c, hw).astype(o_ref.dtype)


def kernel(x, rate=0.1):
    b, c, h, w = x.shape
    hw = h * w
    k = int(round(rate * c))
    if k <= 0:
        return x
    x2 = x.reshape(b, c, hw)
    nb = 4 if b % 4 == 0 else 1
    tri = jnp.asarray(np.tri(c, c, -1, dtype=np.float32))
    bh = (np.abs(np.arange(h)[:, None] - np.arange(h)[None, :]) <= 1)
    bw = (np.abs(np.arange(w)[:, None] - np.arange(w)[None, :]) <= 1)
    band = jnp.asarray(np.kron(bh, bw).astype(np.float32),
                       dtype=jnp.bfloat16)
    out2 = pl.pallas_call(
        lambda tri_ref, band_ref, x_ref, o_ref: _fused_kernel(
            w, k, nb, c, tri_ref, band_ref, x_ref, o_ref),
        out_shape=jax.ShapeDtypeStruct((b, c, hw), x.dtype),
        grid=(b // nb,),
        in_specs=[pl.BlockSpec((c, c), lambda i: (0, 0)),
                  pl.BlockSpec((hw, hw), lambda i: (0, 0)),
                  pl.BlockSpec((nb, c, hw), lambda i: (i, 0, 0))],
        out_specs=pl.BlockSpec((nb, c, hw), lambda i: (i, 0, 0)),
        compiler_params=pltpu.CompilerParams(
            dimension_semantics=("arbitrary",),
            vmem_limit_bytes=_VMEM_LIMIT),
    )(tri, band, x2)
    return out2.reshape(b, c, h, w)
